# baseline (device time: 48408 ns/iter reference)
import jax
import jax.numpy as jnp
from jax import lax
from jax.experimental import pallas as pl
from jax.experimental.pallas import tpu as pltpu

N_DEV = 4
B, SQ, SKV, HQ_LOCAL, DH = 2, 256, 256, 4, 64
DMODEL = 512
WINDOW = 128
SCALE = 0.125


def kernel(x, Wq, K_ext, V_ext, Wo):
    my = lax.axis_index("i")
    K_sh = lax.dynamic_slice_in_dim(K_ext, my * HQ_LOCAL, HQ_LOCAL, axis=2)
    V_sh = lax.dynamic_slice_in_dim(V_ext, my * HQ_LOCAL, HQ_LOCAL, axis=2)
    K_sh = jnp.transpose(K_sh, (0, 2, 1, 3))
    V_sh = jnp.transpose(V_sh, (0, 2, 1, 3))

    def body(x_ref, wq_ref, k_ref, v_ref, wo_ref, out_ref,
             ctx_ref, comm_ref, send_sems, recv_sems):
        my_pos = lax.axis_index("i")
        left = (my_pos - 1) % N_DEV
        right = (my_pos + 1) % N_DEV

        barrier_sem = pltpu.get_barrier_semaphore()
        for nbr in [left, right]:
            pl.semaphore_signal(
                barrier_sem, inc=1,
                device_id=(nbr,), device_id_type=pl.DeviceIdType.MESH,
            )
        pl.semaphore_wait(barrier_sem, 2)

        qi = lax.broadcasted_iota(jnp.int32, (SQ, SKV), 0)
        ki = lax.broadcasted_iota(jnp.int32, (SQ, SKV), 1)
        mask = jnp.abs(qi - ki) <= WINDOW

        wq = wq_ref[:, :]
        for b in range(B):
            q_b = jnp.dot(x_ref[b], wq, preferred_element_type=jnp.float32)
            for h in range(HQ_LOCAL):
                q_bh = q_b[:, h * DH:(h + 1) * DH]
                k_bh = k_ref[b, h]
                s = lax.dot_general(
                    q_bh, k_bh, (((1,), (1,)), ((), ())),
                    preferred_element_type=jnp.float32,
                ) * SCALE
                s = jnp.where(mask, s, -1e9)
                m = jnp.max(s, axis=1, keepdims=True)
                w = jnp.exp(s - m)
                w = w / jnp.sum(w, axis=1, keepdims=True)
                ctx_ref[b, :, h * DH:(h + 1) * DH] = jnp.dot(
                    w, v_ref[b, h], preferred_element_type=jnp.float32
                )
        wo = wo_ref[:, :]
        for b in range(B):
            p_b = jnp.dot(ctx_ref[b], wo, preferred_element_type=jnp.float32)
            out_ref[b] = p_b
            comm_ref[0, b] = p_b

        for h in range(N_DEV - 1):
            send_slot = h % 2
            recv_slot = (h + 1) % 2
            rdma = pltpu.make_async_remote_copy(
                src_ref=comm_ref.at[send_slot],
                dst_ref=comm_ref.at[recv_slot],
                send_sem=send_sems.at[send_slot],
                recv_sem=recv_sems.at[recv_slot],
                device_id=(right,),
                device_id_type=pl.DeviceIdType.MESH,
            )
            rdma.start()
            rdma.wait()
            out_ref[:] = out_ref[:] + comm_ref[recv_slot]

    return pl.pallas_call(
        body,
        out_shape=jax.ShapeDtypeStruct((B, SQ, DMODEL), jnp.float32),
        in_specs=[pl.BlockSpec(memory_space=pltpu.VMEM)] * 5,
        out_specs=pl.BlockSpec(memory_space=pltpu.VMEM),
        scratch_shapes=[
            pltpu.VMEM((B, SQ, HQ_LOCAL * DH), jnp.float32),
            pltpu.VMEM((2, B, SQ, DMODEL), jnp.float32),
            pltpu.SemaphoreType.DMA((2,)),
            pltpu.SemaphoreType.DMA((2,)),
        ],
        compiler_params=pltpu.CompilerParams(collective_id=0),
    )(x, Wq, K_sh, V_sh, Wo)


# device time: 31279 ns/iter; 1.5476x vs baseline; 1.5476x over previous
import jax
import jax.numpy as jnp
from jax import lax
from jax.experimental import pallas as pl
from jax.experimental.pallas import tpu as pltpu

N_DEV = 4
B, SQ, SKV, HQ_LOCAL, DH = 2, 256, 256, 4, 64
DMODEL = 512
WINDOW = 128
SCALE = 0.125


def kernel(x, Wq, K_ext, V_ext, Wo):
    my = lax.axis_index("i")
    K_sh = lax.dynamic_slice_in_dim(K_ext, my * HQ_LOCAL, HQ_LOCAL, axis=2)
    V_sh = lax.dynamic_slice_in_dim(V_ext, my * HQ_LOCAL, HQ_LOCAL, axis=2)
    K_sh = jnp.transpose(K_sh, (0, 2, 1, 3))
    V_sh = jnp.transpose(V_sh, (0, 2, 1, 3))

    def body(x_ref, wq_ref, k_ref, v_ref, wo_ref, out_ref,
             ctx_ref, comm_r, comm_l,
             send_sems_r, recv_sems_r, send_sems_l, recv_sems_l):
        my_pos = lax.axis_index("i")
        left = (my_pos - 1) % N_DEV
        right = (my_pos + 1) % N_DEV

        barrier_sem = pltpu.get_barrier_semaphore()
        for nbr in [left, right]:
            pl.semaphore_signal(
                barrier_sem, inc=1,
                device_id=(nbr,), device_id_type=pl.DeviceIdType.MESH,
            )
        pl.semaphore_wait(barrier_sem, 2)

        qi = lax.broadcasted_iota(jnp.int32, (SQ, SKV), 0)
        ki = lax.broadcasted_iota(jnp.int32, (SQ, SKV), 1)
        mask = jnp.abs(qi - ki) <= WINDOW

        wq = wq_ref[:, :]
        for b in range(B):
            q_b = jnp.dot(x_ref[b], wq, preferred_element_type=jnp.float32)
            for h in range(HQ_LOCAL):
                q_bh = q_b[:, h * DH:(h + 1) * DH]
                k_bh = k_ref[b, h]
                s = lax.dot_general(
                    q_bh, k_bh, (((1,), (1,)), ((), ())),
                    preferred_element_type=jnp.float32,
                ) * SCALE
                w = jnp.where(mask, jnp.exp(s), 0.0)
                w = w / jnp.sum(w, axis=1, keepdims=True)
                ctx_ref[b, :, h * DH:(h + 1) * DH] = jnp.dot(
                    w, v_ref[b, h], preferred_element_type=jnp.float32
                )
        half = DMODEL // 2
        wo = wo_ref[:, :]
        for b in range(B):
            p_b = jnp.dot(ctx_ref[b], wo, preferred_element_type=jnp.float32)
            out_ref[b] = p_b
            comm_r[0, b] = p_b[:, :half]
            comm_l[0, b] = p_b[:, half:]

        for h in range(N_DEV - 1):
            send_slot = h % 2
            recv_slot = (h + 1) % 2
            rdma_r = pltpu.make_async_remote_copy(
                src_ref=comm_r.at[send_slot],
                dst_ref=comm_r.at[recv_slot],
                send_sem=send_sems_r.at[send_slot],
                recv_sem=recv_sems_r.at[recv_slot],
                device_id=(right,),
                device_id_type=pl.DeviceIdType.MESH,
            )
            rdma_l = pltpu.make_async_remote_copy(
                src_ref=comm_l.at[send_slot],
                dst_ref=comm_l.at[recv_slot],
                send_sem=send_sems_l.at[send_slot],
                recv_sem=recv_sems_l.at[recv_slot],
                device_id=(left,),
                device_id_type=pl.DeviceIdType.MESH,
            )
            rdma_r.start()
            rdma_l.start()
            rdma_r.wait()
            rdma_l.wait()
            out_ref[:, :, :half] = out_ref[:, :, :half] + comm_r[recv_slot]
            out_ref[:, :, half:] = out_ref[:, :, half:] + comm_l[recv_slot]

    return pl.pallas_call(
        body,
        out_shape=jax.ShapeDtypeStruct((B, SQ, DMODEL), jnp.float32),
        in_specs=[pl.BlockSpec(memory_space=pltpu.VMEM)] * 5,
        out_specs=pl.BlockSpec(memory_space=pltpu.VMEM),
        scratch_shapes=[
            pltpu.VMEM((B, SQ, HQ_LOCAL * DH), jnp.float32),
            pltpu.VMEM((2, B, SQ, DMODEL // 2), jnp.float32),
            pltpu.VMEM((2, B, SQ, DMODEL // 2), jnp.float32),
            pltpu.SemaphoreType.DMA((2,)),
            pltpu.SemaphoreType.DMA((2,)),
            pltpu.SemaphoreType.DMA((2,)),
            pltpu.SemaphoreType.DMA((2,)),
        ],
        compiler_params=pltpu.CompilerParams(collective_id=0),
    )(x, Wq, K_sh, V_sh, Wo)


# device time: 24255 ns/iter; 1.9958x vs baseline; 1.2896x over previous
import jax
import jax.numpy as jnp
from jax import lax
from jax.experimental import pallas as pl
from jax.experimental.pallas import tpu as pltpu

N_DEV = 4
B, SQ, SKV, HQ_LOCAL, DH = 2, 256, 256, 4, 64
DMODEL = 512
WINDOW = 128
SCALE = 0.125


def kernel(x, Wq, K_ext, V_ext, Wo):
    my = lax.axis_index("i")
    K_sh = lax.dynamic_slice_in_dim(K_ext, my * HQ_LOCAL, HQ_LOCAL, axis=2)
    V_sh = lax.dynamic_slice_in_dim(V_ext, my * HQ_LOCAL, HQ_LOCAL, axis=2)
    K_sh = jnp.transpose(K_sh, (0, 2, 1, 3))
    V_sh = jnp.transpose(V_sh, (0, 2, 1, 3))

    def body(x_ref, wq_ref, k_ref, v_ref, wo_ref, out_ref,
             ctx_ref, scat_ref, send_a, recv_a, send_b, recv_b):
        me = lax.axis_index("i")

        barrier_sem = pltpu.get_barrier_semaphore()
        for d in range(1, N_DEV):
            pl.semaphore_signal(
                barrier_sem, inc=1,
                device_id=((me + d) % N_DEV,),
                device_id_type=pl.DeviceIdType.MESH,
            )
        pl.semaphore_wait(barrier_sem, N_DEV - 1)

        qi = lax.broadcasted_iota(jnp.int32, (SQ, SKV), 0)
        ki = lax.broadcasted_iota(jnp.int32, (SQ, SKV), 1)
        mask = jnp.abs(qi - ki) <= WINDOW

        wq = wq_ref[:, :]
        for b in range(B):
            q_b = jnp.dot(x_ref[b], wq, preferred_element_type=jnp.float32)
            for h in range(HQ_LOCAL):
                q_bh = q_b[:, h * DH:(h + 1) * DH]
                k_bh = k_ref[b, h]
                s = lax.dot_general(
                    q_bh, k_bh, (((1,), (1,)), ((), ())),
                    preferred_element_type=jnp.float32,
                ) * SCALE
                w = jnp.where(mask, jnp.exp(s), 0.0)
                w = w / jnp.sum(w, axis=1, keepdims=True)
                ctx_ref[b, :, h * DH:(h + 1) * DH] = jnp.dot(
                    w, v_ref[b, h], preferred_element_type=jnp.float32
                )
        wo = wo_ref[:, :]
        for b in range(B):
            out_ref[b] = jnp.dot(ctx_ref[b], wo, preferred_element_type=jnp.float32)

        QR = SQ // N_DEV

        sends_a = []
        for d in range(1, N_DEV):
            t = (me + d) % N_DEV
            rdma = pltpu.make_async_remote_copy(
                src_ref=out_ref.at[:, pl.ds(t * QR, QR), :],
                dst_ref=scat_ref.at[N_DEV - 1 - d],
                send_sem=send_a.at[d - 1],
                recv_sem=recv_a.at[N_DEV - 1 - d],
                device_id=(t,),
                device_id_type=pl.DeviceIdType.MESH,
            )
            rdma.start()
            sends_a.append(rdma)
        for r in range(N_DEV - 1):
            pltpu.make_async_remote_copy(
                src_ref=scat_ref.at[r], dst_ref=scat_ref.at[r],
                send_sem=send_a.at[r], recv_sem=recv_a.at[r],
                device_id=(me,), device_id_type=pl.DeviceIdType.MESH,
            ).wait_recv()
        for rdma in sends_a:
            rdma.wait_send()

        my_rows = pl.ds(me * QR, QR)
        out_ref[:, my_rows, :] = (
            out_ref[:, my_rows, :] + scat_ref[0] + scat_ref[1] + scat_ref[2]
        )

        sends_b = []
        for d in range(1, N_DEV):
            t = (me + d) % N_DEV
            rdma = pltpu.make_async_remote_copy(
                src_ref=out_ref.at[:, my_rows, :],
                dst_ref=out_ref.at[:, my_rows, :],
                send_sem=send_b.at[d - 1],
                recv_sem=recv_b.at[N_DEV - 1 - d],
                device_id=(t,),
                device_id_type=pl.DeviceIdType.MESH,
            )
            rdma.start()
            sends_b.append(rdma)
        for r in range(N_DEV - 1):
            s = (me + 1 + r) % N_DEV
            pltpu.make_async_remote_copy(
                src_ref=out_ref.at[:, pl.ds(s * QR, QR), :],
                dst_ref=out_ref.at[:, pl.ds(s * QR, QR), :],
                send_sem=send_b.at[r], recv_sem=recv_b.at[r],
                device_id=(me,), device_id_type=pl.DeviceIdType.MESH,
            ).wait_recv()
        for rdma in sends_b:
            rdma.wait_send()

    return pl.pallas_call(
        body,
        out_shape=jax.ShapeDtypeStruct((B, SQ, DMODEL), jnp.float32),
        in_specs=[pl.BlockSpec(memory_space=pltpu.VMEM)] * 5,
        out_specs=pl.BlockSpec(memory_space=pltpu.VMEM),
        scratch_shapes=[
            pltpu.VMEM((B, SQ, HQ_LOCAL * DH), jnp.float32),
            pltpu.VMEM((N_DEV - 1, B, SQ // N_DEV, DMODEL), jnp.float32),
            pltpu.SemaphoreType.DMA((N_DEV - 1,)),
            pltpu.SemaphoreType.DMA((N_DEV - 1,)),
            pltpu.SemaphoreType.DMA((N_DEV - 1,)),
            pltpu.SemaphoreType.DMA((N_DEV - 1,)),
        ],
        compiler_params=pltpu.CompilerParams(collective_id=0),
    )(x, Wq, K_sh, V_sh, Wo)
